# baseline (speedup 1.0000x reference)
import functools

import jax
import jax.numpy as jnp
from jax import lax
from jax.experimental import pallas as pl
from jax.experimental.pallas import tpu as pltpu
from jax.experimental.pallas import tpu_sc as plsc

_info = plsc.get_sparse_core_info()
_NC = _info.num_cores
_NS = _info.num_subcores
_NW = _NC * _NS

_SLAB = 80
_LANE = 16


def _make_kernel(batch, seq, hidden):
    n_rows = batch * seq
    rows_per_w = n_rows // _NW
    n_slabs = rows_per_w // _SLAB
    reps = _SLAB // seq
    cchunks = hidden // _LANE

    mesh = plsc.VectorSubcoreMesh(core_axis_name="c", subcore_axis_name="s")

    @functools.partial(
        pl.kernel,
        mesh=mesh,
        compiler_params=pltpu.CompilerParams(use_tc_tiling_on_sc=False),
        out_type=jax.ShapeDtypeStruct((n_rows, hidden), jnp.float32),
        scratch_types=[
            pltpu.VMEM((n_slabs, _SLAB), jnp.int32),
            pltpu.VMEM((seq, hidden), jnp.float32),
            pltpu.VMEM((2, _SLAB, hidden), jnp.float32),
            pltpu.SemaphoreType.DMA,
            pltpu.SemaphoreType.DMA,
        ],
    )
    def emb_kernel(x_hbm, table_hbm, pos_hbm, out_hbm, idx_v, pos_v, buf_v, gsem, osem):
        wid = lax.axis_index("s") * _NC + lax.axis_index("c")
        pltpu.sync_copy(x_hbm.at[wid], idx_v)
        pltpu.sync_copy(pos_hbm, pos_v)
        base = wid * rows_per_w

        def gather(s, plane):
            return pltpu.make_async_copy(
                table_hbm.at[idx_v.at[s]], buf_v.at[plane], gsem)

        def out_copy(s, plane):
            return pltpu.make_async_copy(
                buf_v.at[plane], out_hbm.at[pl.ds(base + s * _SLAB, _SLAB)], osem)

        def add_pos(plane):
            for c in range(cchunks):
                sl = pl.ds(c * _LANE, _LANE)
                for p in range(seq):
                    pv = pos_v[p, sl]
                    for r in range(reps):
                        row = r * seq + p
                        buf_v[plane, row, sl] = buf_v[plane, row, sl] + pv

        gather(0, 0).start()

        def body(g, _):
            s0 = 2 * g
            gather(s0, 0).wait()

            @pl.when(g >= 1)
            def _():
                out_copy(s0 - 1, 1).wait()

            gather(s0 + 1, 1).start()
            add_pos(0)
            out_copy(s0, 0).start()

            gather(s0 + 1, 1).wait()
            out_copy(s0, 0).wait()

            @pl.when(g < n_slabs // 2 - 1)
            def _():
                gather(s0 + 2, 0).start()

            add_pos(1)
            out_copy(s0 + 1, 1).start()
            return ()

        lax.fori_loop(0, n_slabs // 2, body, (), unroll=False)
        out_copy(n_slabs - 1, 1).wait()

    return emb_kernel


def kernel(x, table, pos_emb):
    batch, seq = x.shape
    hidden = table.shape[1]
    xr = x.reshape(_NW, (batch * seq) // (_NW * _SLAB), _SLAB)
    pos = pos_emb[0, :seq, :]
    out = _make_kernel(batch, seq, hidden)(xr, table, pos)
    return out.reshape(batch, seq, hidden)
